# baseline (speedup 1.0000x reference)
import jax
import jax.numpy as jnp
from jax import lax
from jax.experimental import pallas as pl
from jax.experimental.pallas import tpu as pltpu
from jax.experimental.pallas import tpu_sc as plsc

_PAD = 2
_DW = 64
_DP = 16
_DOUT = _DW + 2 * _DP
_NC = 2
_NS = 16
_NW = _NC * _NS
_TP = 204
_TS = 208
_G0 = 96
_G1 = 112
_SUP = 16
_SL = _SUP * _TS


def _lane(v, l):
    return jnp.take_along_axis(v, jnp.full((16,), l, jnp.int32), axis=0)


def _gather_concat(we2, wpe_flat, ipair, wmix, b):
    ipw = b // _NW
    nchunks = ipw // _SUP
    mesh = plsc.VectorSubcoreMesh(core_axis_name="core", subcore_axis_name="subcore")

    @pl.kernel(
        out_type=jax.ShapeDtypeStruct((b, _TP, _DOUT), jnp.float32),
        mesh=mesh,
        compiler_params=pltpu.CompilerParams(needs_layout_passes=False),
        scratch_types=[
            pltpu.VMEM((_SL,), jnp.int32),
            pltpu.VMEM((_SL,), jnp.int32),
            pltpu.VMEM((_SL,), jnp.int32),
            pltpu.VMEM((_SL,), jnp.int32),
            pltpu.VMEM((_G0, 2 * _DW), jnp.float32),
            pltpu.VMEM((_G0, 2 * _DW), jnp.float32),
            pltpu.VMEM((_G1, 2 * _DW), jnp.float32),
            pltpu.VMEM((_G1, 2 * _DW), jnp.float32),
            pltpu.VMEM((_TP, _DOUT), jnp.float32),
            pltpu.VMEM((_TP, _DOUT), jnp.float32),
            pltpu.VMEM((400 * _DP,), jnp.float32),
            pltpu.SemaphoreType.DMA,
            pltpu.SemaphoreType.DMA,
            pltpu.SemaphoreType.DMA,
            pltpu.SemaphoreType.DMA,
            pltpu.SemaphoreType.DMA,
        ],
    )
    def k(we_hbm, wpe_hbm, ip_hbm, wm_hbm, o_hbm,
          vipA, vipB, vmixA, vmixB, bp0A, bp0B, bp1A, bp1B, b96A, b96B,
          wpv, g0s, g1s, w0s, w1s, ssem):
        wid = lax.axis_index("core") * _NS + lax.axis_index("subcore")
        item0 = wid * ipw
        pltpu.sync_copy(wpe_hbm, wpv)
        iota = lax.iota(jnp.int32, 16)
        gs = (g0s, g1s)
        ws = (w0s, w1s)
        vip = (vipA, vipB)
        vmix = (vmixA, vmixB)
        bp0 = (bp0A, bp0B)
        bp1 = (bp1A, bp1B)
        b96 = (b96A, b96B)

        def issue_gathers(vipref, loc, p):
            rbn = loc * _TS
            pltpu.async_copy(
                we_hbm.at[vipref.at[pl.ds(rbn, _G0)]], bp0[p], gs[p])
            pltpu.async_copy(
                we_hbm.at[vipref.at[pl.ds(rbn + _G0, _G1)]], bp1[p], gs[p])

        def wait_gathers(p):
            pltpu.make_async_copy(
                we_hbm.at[pl.ds(0, _G0)], bp0[p], gs[p]).wait()
            pltpu.make_async_copy(
                we_hbm.at[pl.ds(0, _G1)], bp1[p], gs[p]).wait()

        def stage_chunk(c, qs):
            nbase = (item0 + c * _SUP) * _TS
            pltpu.async_copy(ip_hbm.at[pl.ds(nbase, _SL)], vip[qs], ssem)
            pltpu.async_copy(wm_hbm.at[pl.ds(nbase, _SL)], vmix[qs], ssem)

        def wait_stage(qs):
            pltpu.make_async_copy(
                ip_hbm.at[pl.ds(0, _SL)], vip[qs], ssem).wait()
            pltpu.make_async_copy(
                wm_hbm.at[pl.ds(0, _SL)], vmix[qs], ssem).wait()

        def emit_block(vmixref, rb, dst_t0, src, dst, src_t0, nrows):
            wv = vmixref[pl.ds(rb + dst_t0, 16)]
            offv = jnp.right_shift(wv, 25) & 64
            o1v = wv & 0xFFFF
            o2v = jnp.right_shift(wv, 16) & 0x7FFF
            for l in range(nrows):
                rowc = jnp.full((16,), src_t0 + l, jnp.int32)
                colb = _lane(offv, l) + iota
                for cc in range(4):
                    dst[dst_t0 + l, pl.ds(cc * 16, 16)] = plsc.load_gather(
                        src, [rowc, colb + cc * 16])
                dst[dst_t0 + l, pl.ds(_DW, _DP)] = plsc.load_gather(
                    wpv, [_lane(o1v, l) + iota])
                dst[dst_t0 + l, pl.ds(_DW + _DP, _DP)] = plsc.load_gather(
                    wpv, [_lane(o2v, l) + iota])

        def assemble(vmixref, loc, p):
            rb = loc * _TS

            @pl.loop(0, _G0 // 16)
            def _rows_lo(blk):
                t0 = blk * 16
                emit_block(vmixref, rb, t0, bp0[p], b96[p], t0, 16)

            @pl.loop(0, 6)
            def _rows_hi(blk):
                t0 = blk * 16
                emit_block(vmixref, rb, _G0 + t0, bp1[p], b96[p], t0, 16)

            emit_block(vmixref, rb, 192, bp1[p], b96[p], 96, 12)

        stage_chunk(0, 0)
        wait_stage(0)
        issue_gathers(vipA, 0, 0)

        @pl.loop(0, nchunks // 2)
        def _(cp):
            for qs in (0, 1):
                c = 2 * cp + qs

                @pl.loop(0, _SUP // 2)
                def _(jj):
                    for p in (0, 1):
                        loc = 2 * jj + p
                        i = c * _SUP + loc

                        if p == 0:
                            @pl.when((jj == 0) & (c + 1 < nchunks))
                            def _prefetch():
                                stage_chunk(c + 1, 1 - qs)

                        wait_gathers(p)

                        if p == 0:
                            issue_gathers(vip[qs], loc + 1, 1 - p)
                        else:
                            @pl.when(jj < _SUP // 2 - 1)
                            def _issue_same_chunk():
                                issue_gathers(vip[qs], loc + 1, 1 - p)

                            @pl.when((jj == _SUP // 2 - 1) & (c + 1 < nchunks))
                            def _issue_next_chunk():
                                wait_stage(1 - qs)
                                issue_gathers(vip[1 - qs], 0, 1 - p)

                        @pl.when(i >= 2)
                        def _wait_write():
                            pltpu.make_async_copy(
                                b96[p], o_hbm.at[item0 + i - 2], ws[p]).wait()

                        assemble(vmix[qs], loc, p)
                        pltpu.async_copy(b96[p], o_hbm.at[item0 + i], ws[p])

        pltpu.make_async_copy(b96[0], o_hbm.at[item0 + ipw - 2], w0s).wait()
        pltpu.make_async_copy(b96[1], o_hbm.at[item0 + ipw - 1], w1s).wait()

    return k(we2, wpe_flat, ipair, wmix)


def kernel(seq_inputs, e1_pos_inputs, e2_pos_inputs, we, wpe):
    b, t = seq_inputs.shape

    si = seq_inputs.astype(jnp.int32)
    e1 = e1_pos_inputs.astype(jnp.int32)
    e2 = e2_pos_inputs.astype(jnp.int32)

    zpad2 = jnp.zeros((b, _PAD), jnp.int32)
    ztail = jnp.zeros((b, _TS - _TP), jnp.int32)
    si_p = jnp.concatenate([zpad2, si, zpad2, ztail], axis=1)

    def edge_pad(x):
        head = jnp.repeat(x[:, :1], _PAD, axis=1)
        tail = jnp.repeat(x[:, -1:], _PAD, axis=1)
        return jnp.concatenate([head, x, tail, ztail], axis=1)

    e1_p = edge_pad(e1)
    e2_p = edge_pad(e2)

    ipair = (si_p >> 1).reshape(-1)
    wmix = (((si_p & 1) << 31) | ((e2_p * _DP) << 16) | (e1_p * _DP)).reshape(-1)

    out = _gather_concat(we.reshape(we.shape[0] // 2, 2 * _DW),
                         wpe.reshape(-1), ipair, wmix, b)
    return out


# speedup vs baseline: 1.0272x; 1.0272x over previous
import jax
import jax.numpy as jnp
from jax import lax
from jax.experimental import pallas as pl
from jax.experimental.pallas import tpu as pltpu
from jax.experimental.pallas import tpu_sc as plsc

_PAD = 2
_DW = 64
_DP = 16
_DOUT = _DW + 2 * _DP
_NC = 2
_NS = 16
_NW = _NC * _NS
_TP = 204
_TS = 208
_G0 = 96
_G1 = 112
_SUP = 8
_SL = _SUP * _TS


def _lane(v, l):
    return jnp.take_along_axis(v, jnp.full((16,), l, jnp.int32), axis=0)


def _gather_concat(we2, wpe_flat, ipair, wmix, b):
    ipw = b // _NW
    nchunks = ipw // _SUP
    mesh = plsc.VectorSubcoreMesh(core_axis_name="core", subcore_axis_name="subcore")

    @pl.kernel(
        out_type=jax.ShapeDtypeStruct((b, _TS, 2 * _DW), jnp.float32),
        mesh=mesh,
        compiler_params=pltpu.CompilerParams(needs_layout_passes=False),
        scratch_types=[
            pltpu.VMEM((_SL,), jnp.int32),
            pltpu.VMEM((_SL,), jnp.int32),
            pltpu.VMEM((_SL,), jnp.int32),
            pltpu.VMEM((_SL,), jnp.int32),
            pltpu.VMEM((_G0, 2 * _DW), jnp.float32),
            pltpu.VMEM((_G0, 2 * _DW), jnp.float32),
            pltpu.VMEM((_G1, 2 * _DW), jnp.float32),
            pltpu.VMEM((_G1, 2 * _DW), jnp.float32),
            pltpu.VMEM((_TS, 2 * _DW), jnp.float32),
            pltpu.VMEM((_TS, 2 * _DW), jnp.float32),
            pltpu.VMEM((400 * _DP,), jnp.float32),
            pltpu.SemaphoreType.DMA,
            pltpu.SemaphoreType.DMA,
            pltpu.SemaphoreType.DMA,
            pltpu.SemaphoreType.DMA,
            pltpu.SemaphoreType.DMA,
        ],
    )
    def k(we_hbm, wpe_hbm, ip_hbm, wm_hbm, o_hbm,
          vipA, vipB, vmixA, vmixB, bp0A, bp0B, bp1A, bp1B, b96A, b96B,
          wpv, g0s, g1s, w0s, w1s, ssem):
        wid = lax.axis_index("core") * _NS + lax.axis_index("subcore")
        item0 = wid * ipw
        pltpu.sync_copy(wpe_hbm, wpv)
        iota = lax.iota(jnp.int32, 16)
        gs = (g0s, g1s)
        ws = (w0s, w1s)
        vip = (vipA, vipB)
        vmix = (vmixA, vmixB)
        bp0 = (bp0A, bp0B)
        bp1 = (bp1A, bp1B)
        b96 = (b96A, b96B)

        def issue_gathers(vipref, loc, p):
            rbn = loc * _TS
            pltpu.async_copy(
                we_hbm.at[vipref.at[pl.ds(rbn, _G0)]], bp0[p], gs[p])
            pltpu.async_copy(
                we_hbm.at[vipref.at[pl.ds(rbn + _G0, _G1)]], bp1[p], gs[p])

        def wait_gathers(p):
            pltpu.make_async_copy(
                we_hbm.at[pl.ds(0, _G0)], bp0[p], gs[p]).wait()
            pltpu.make_async_copy(
                we_hbm.at[pl.ds(0, _G1)], bp1[p], gs[p]).wait()

        def stage_chunk(c, qs):
            nbase = (item0 + c * _SUP) * _TS
            pltpu.async_copy(ip_hbm.at[pl.ds(nbase, _SL)], vip[qs], ssem)
            pltpu.async_copy(wm_hbm.at[pl.ds(nbase, _SL)], vmix[qs], ssem)

        def wait_stage(qs):
            pltpu.make_async_copy(
                ip_hbm.at[pl.ds(0, _SL)], vip[qs], ssem).wait()
            pltpu.make_async_copy(
                wm_hbm.at[pl.ds(0, _SL)], vmix[qs], ssem).wait()

        def emit_block(vmixref, rb, dst_t0, src, dst, src_t0, nrows):
            wv = vmixref[pl.ds(rb + dst_t0, 16)]
            offv = jnp.right_shift(wv, 25) & 64
            o1v = wv & 0xFFFF
            o2v = jnp.right_shift(wv, 16) & 0x7FFF
            for l in range(nrows):
                rowc = jnp.full((16,), src_t0 + l, jnp.int32)
                colb = _lane(offv, l) + iota
                for cc in range(4):
                    dst[dst_t0 + l, pl.ds(cc * 16, 16)] = plsc.load_gather(
                        src, [rowc, colb + cc * 16])
                dst[dst_t0 + l, pl.ds(_DW, _DP)] = plsc.load_gather(
                    wpv, [_lane(o1v, l) + iota])
                dst[dst_t0 + l, pl.ds(_DW + _DP, _DP)] = plsc.load_gather(
                    wpv, [_lane(o2v, l) + iota])

        def assemble(vmixref, loc, p):
            rb = loc * _TS

            @pl.loop(0, _G0 // 16)
            def _rows_lo(blk):
                t0 = blk * 16
                emit_block(vmixref, rb, t0, bp0[p], b96[p], t0, 16)

            @pl.loop(0, 6)
            def _rows_hi(blk):
                t0 = blk * 16
                emit_block(vmixref, rb, _G0 + t0, bp1[p], b96[p], t0, 16)

            emit_block(vmixref, rb, 192, bp1[p], b96[p], 96, 12)

        stage_chunk(0, 0)
        wait_stage(0)
        issue_gathers(vipA, 0, 0)

        @pl.loop(0, nchunks // 2)
        def _(cp):
            for qs in (0, 1):
                c = 2 * cp + qs

                @pl.loop(0, _SUP // 2)
                def _(jj):
                    for p in (0, 1):
                        loc = 2 * jj + p
                        i = c * _SUP + loc

                        if p == 0:
                            @pl.when((jj == 0) & (c + 1 < nchunks))
                            def _prefetch():
                                stage_chunk(c + 1, 1 - qs)

                        wait_gathers(p)

                        if p == 0:
                            issue_gathers(vip[qs], loc + 1, 1 - p)
                        else:
                            @pl.when(jj < _SUP // 2 - 1)
                            def _issue_same_chunk():
                                issue_gathers(vip[qs], loc + 1, 1 - p)

                            @pl.when((jj == _SUP // 2 - 1) & (c + 1 < nchunks))
                            def _issue_next_chunk():
                                wait_stage(1 - qs)
                                issue_gathers(vip[1 - qs], 0, 1 - p)

                        @pl.when(i >= 2)
                        def _wait_write():
                            pltpu.make_async_copy(
                                b96[p], o_hbm.at[item0 + i - 2], ws[p]).wait()

                        assemble(vmix[qs], loc, p)
                        pltpu.async_copy(b96[p], o_hbm.at[item0 + i], ws[p])

        pltpu.make_async_copy(b96[0], o_hbm.at[item0 + ipw - 2], w0s).wait()
        pltpu.make_async_copy(b96[1], o_hbm.at[item0 + ipw - 1], w1s).wait()

    return k(we2, wpe_flat, ipair, wmix)


def kernel(seq_inputs, e1_pos_inputs, e2_pos_inputs, we, wpe):
    b, t = seq_inputs.shape

    si = seq_inputs.astype(jnp.int32)
    e1 = e1_pos_inputs.astype(jnp.int32)
    e2 = e2_pos_inputs.astype(jnp.int32)

    zpad2 = jnp.zeros((b, _PAD), jnp.int32)
    ztail = jnp.zeros((b, _TS - _TP), jnp.int32)
    si_p = jnp.concatenate([zpad2, si, zpad2, ztail], axis=1)

    def edge_pad(x):
        head = jnp.repeat(x[:, :1], _PAD, axis=1)
        tail = jnp.repeat(x[:, -1:], _PAD, axis=1)
        return jnp.concatenate([head, x, tail, ztail], axis=1)

    e1_p = edge_pad(e1)
    e2_p = edge_pad(e2)

    ipair = (si_p >> 1).reshape(-1)
    wmix = (((si_p & 1) << 31) | ((e2_p * _DP) << 16) | (e1_p * _DP)).reshape(-1)

    out = _gather_concat(we.reshape(we.shape[0] // 2, 2 * _DW),
                         wpe.reshape(-1), ipair, wmix, b)
    return out[:, :_TP, :_DOUT]


# speedup vs baseline: 1.5360x; 1.4953x over previous
import jax
import jax.numpy as jnp
from jax import lax
from jax.experimental import pallas as pl
from jax.experimental.pallas import tpu as pltpu
from jax.experimental.pallas import tpu_sc as plsc

_PAD = 2
_DW = 64
_DP = 16
_DOUT = _DW + 2 * _DP
_W = 128
_NC = 2
_NS = 16
_NW = _NC * _NS


def _gather_concat(we, wpe, si, e1, e2, rows):
    rpw = rows // _NW
    nwin = rpw // _W
    mesh = plsc.VectorSubcoreMesh(core_axis_name="core", subcore_axis_name="subcore")

    @pl.kernel(
        out_type=jax.ShapeDtypeStruct((rows, _DOUT), jnp.float32),
        mesh=mesh,
        compiler_params=pltpu.CompilerParams(use_tc_tiling_on_sc=False),
        scratch_types=[
            pltpu.VMEM((nwin, _W), jnp.int32),
            pltpu.VMEM((nwin, _W), jnp.int32),
            pltpu.VMEM((nwin, _W), jnp.int32),
            pltpu.VMEM((_W, _DW), jnp.float32),
            pltpu.VMEM((_W, _DW), jnp.float32),
            pltpu.VMEM((_W, _DP), jnp.float32),
            pltpu.VMEM((_W, _DP), jnp.float32),
            pltpu.VMEM((_W, _DP), jnp.float32),
            pltpu.VMEM((_W, _DP), jnp.float32),
            pltpu.VMEM_SHARED((_NS * 2, _W, _DOUT), jnp.float32),
            pltpu.SemaphoreType.DMA,
            pltpu.SemaphoreType.DMA,
            pltpu.SemaphoreType.DMA,
            pltpu.SemaphoreType.DMA,
            pltpu.SemaphoreType.DMA,
        ],
    )
    def k(we_hbm, wpe_hbm, si_hbm, e1_hbm, e2_hbm, o_hbm,
          isi, ie1, ie2, bweA, bweB, be1A, be1B, be2A, be2B, shb,
          g0s, g1s, asem, w0s, w1s):
        sid = lax.axis_index("subcore")
        wid = lax.axis_index("core") * _NS + sid
        base0 = wid * rpw
        bwe = (bweA, bweB)
        be1 = (be1A, be1B)
        be2 = (be2A, be2B)
        gs = (g0s, g1s)
        ws = (w0s, w1s)
        c0 = pltpu.async_copy(si_hbm.at[wid], isi, g0s)
        c1 = pltpu.async_copy(e1_hbm.at[wid], ie1, g0s)
        c2 = pltpu.async_copy(e2_hbm.at[wid], ie2, g0s)
        c0.wait(); c1.wait(); c2.wait()

        def issue_g(j, p):
            pltpu.async_copy(we_hbm.at[isi.at[j]], bwe[p], gs[p])
            pltpu.async_copy(wpe_hbm.at[ie1.at[j]], be1[p], gs[p])
            pltpu.async_copy(wpe_hbm.at[ie2.at[j]], be2[p], gs[p])

        def wait_g(p):
            pltpu.make_async_copy(we_hbm.at[pl.ds(0, _W)], bwe[p], gs[p]).wait()
            pltpu.make_async_copy(wpe_hbm.at[pl.ds(0, _W)], be1[p], gs[p]).wait()
            pltpu.make_async_copy(wpe_hbm.at[pl.ds(0, _W)], be2[p], gs[p]).wait()

        issue_g(0, 0)

        @pl.loop(0, nwin // 2)
        def _(jj):
            for p in (0, 1):
                j = 2 * jj + p
                mysh = shb.at[sid * 2 + p]
                wait_g(p)

                @pl.when(j + 1 < nwin)
                def _issue_next():
                    issue_g(j + 1, 1 - p)

                @pl.when(j >= 2)
                def _wait_write():
                    pltpu.make_async_copy(
                        mysh, o_hbm.at[pl.ds(base0 + (j - 2) * _W, _W)],
                        ws[p]).wait()

                a0 = pltpu.async_copy(bwe[p], mysh.at[:, pl.ds(0, _DW)], asem)
                a1 = pltpu.async_copy(be1[p], mysh.at[:, pl.ds(_DW, _DP)], asem)
                a2 = pltpu.async_copy(be2[p], mysh.at[:, pl.ds(_DW + _DP, _DP)], asem)
                a0.wait(); a1.wait(); a2.wait()
                pltpu.async_copy(mysh, o_hbm.at[pl.ds(base0 + j * _W, _W)], ws[p])

        pltpu.make_async_copy(
            shb.at[sid * 2], o_hbm.at[pl.ds(base0 + (nwin - 2) * _W, _W)], w0s).wait()
        pltpu.make_async_copy(
            shb.at[sid * 2 + 1], o_hbm.at[pl.ds(base0 + (nwin - 1) * _W, _W)], w1s).wait()

    return k(we, wpe, si, e1, e2)


def kernel(seq_inputs, e1_pos_inputs, e2_pos_inputs, we, wpe):
    b, t = seq_inputs.shape
    tp = t + 2 * _PAD
    rows = b * tp
    rpw = rows // _NW
    nwin = rpw // _W

    si = seq_inputs.astype(jnp.int32)
    e1 = e1_pos_inputs.astype(jnp.int32)
    e2 = e2_pos_inputs.astype(jnp.int32)

    zpad = jnp.zeros((b, _PAD), jnp.int32)
    si_p = jnp.concatenate([zpad, si, zpad], axis=1)

    def edge_pad(x):
        head = jnp.repeat(x[:, :1], _PAD, axis=1)
        tail = jnp.repeat(x[:, -1:], _PAD, axis=1)
        return jnp.concatenate([head, x, tail], axis=1)

    e1_p = edge_pad(e1)
    e2_p = edge_pad(e2)

    out = _gather_concat(
        we, wpe,
        si_p.reshape(_NW, nwin, _W),
        e1_p.reshape(_NW, nwin, _W),
        e2_p.reshape(_NW, nwin, _W),
        rows,
    )
    return out.reshape(b, tp, _DOUT)
